# baseline (speedup 1.0000x reference)
import functools

import jax
import jax.numpy as jnp
import numpy as np
from jax import lax
from jax.experimental import pallas as pl
from jax.experimental.pallas import tpu as pltpu
from jax.experimental.pallas import tpu_sc as plsc

_N = 1024
_ROW = 1280
_NW = 32
_SPW = _N // _NW
_CHUNK = 16
_TBL = 160

def _sc_body(p0_hbm, p1_hbm, p2_hbm, tbl_hbm, par_hbm, out_hbm,
             buf_a, buf_b, tbl_v, par_v, scr_v, out_v, sem_a, sem_b):
    wid = lax.axis_index("s") * 2 + lax.axis_index("c")
    pltpu.sync_copy(tbl_hbm, tbl_v)
    pltpu.sync_copy(par_hbm, par_v)

    n0 = wid * _SPW
    planes = (p0_hbm, p1_hbm, p2_hbm)
    cps_a = [pltpu.async_copy(p.at[pl.ds(n0, 16)],
                              buf_a.at[pl.ds(16 * r, 16)], sem_a)
             for r, p in enumerate(planes)]
    cps_b = [pltpu.async_copy(p.at[pl.ds(n0 + 16, 16)],
                              buf_b.at[pl.ds(16 * r, 16)], sem_b)
             for r, p in enumerate(planes)]

    lane = lax.iota(jnp.int32, 16)

    a0 = par_v[0]
    a1 = par_v[1]
    b0 = par_v[2]
    b1 = par_v[3]
    cm = par_v[4]

    def do_chunk(buf, cc):
        def sample_body(s, _):
            acc = jnp.zeros((16,), jnp.float32)
            for r in range(3):
                off2 = jnp.int32(800 * r) + lane

                def body(g, acc):
                    for h in range(2):
                        v64 = buf[16 * r + s, g, pl.ds(h * 64, 64)]
                        w = plsc.bitcast(v64, jnp.int32)
                        for k in range(4):
                            if k == 0:
                                ti = (w << 4) & 0xFF0
                            else:
                                ti = (w >> (8 * k - 4)) & 0xFF0
                            acc = acc + plsc.load_gather(tbl_v, [ti + off2])
                    return acc

                acc = lax.fori_loop(0, 10, body, acc, unroll=5)
            plsc.store_scatter(scr_v, [s * 17 + lane], acc)
            return 0

        lax.fori_loop(0, 16, sample_body, 0)

        s_acc = jnp.zeros((16,), jnp.float32)
        for p in range(16):
            s_acc = s_acc + plsc.load_gather(scr_v, [lane * 17 + p])

        mean = s_acc / 10.0 + cm
        orow = jnp.int32(16 * cc) + lane
        plsc.store_scatter(out_v, [orow, jnp.zeros((16,), jnp.int32)],
                           mean * a0 + b0)
        plsc.store_scatter(out_v, [orow, jnp.ones((16,), jnp.int32)],
                           mean * a1 + b1)

    for cp in cps_a:
        cp.wait()
    do_chunk(buf_a, 0)
    for cp in cps_b:
        cp.wait()
    do_chunk(buf_b, 1)

    pltpu.sync_copy(out_v, out_hbm.at[pl.ds(wid * _SPW, _SPW)])


@jax.jit
def _run(p0, p1, p2, tbl, par):
    mesh = plsc.VectorSubcoreMesh(core_axis_name="c", subcore_axis_name="s")
    return pl.kernel(
        _sc_body,
        out_type=jax.ShapeDtypeStruct((_N, 2), jnp.float32),
        mesh=mesh,
        compiler_params=pltpu.CompilerParams(use_tc_tiling_on_sc=False,
                                             needs_layout_passes=False),
        scratch_types=[
            pltpu.VMEM((48, 10, 128), jnp.uint8),
            pltpu.VMEM((48, 10, 128), jnp.uint8),
            pltpu.VMEM((_TBL * 16,), jnp.float32),
            pltpu.VMEM((8, 16), jnp.float32),
            pltpu.VMEM((272,), jnp.float32),
            pltpu.VMEM((_SPW, 2), jnp.float32),
            pltpu.SemaphoreType.DMA,
            pltpu.SemaphoreType.DMA,
        ],
    )(p0, p1, p2, tbl, par)


def _plane(images, d):
    t = jnp.transpose(images[:, d], (0, 2, 1))
    return (t * 50.0).astype(jnp.uint8)


def kernel(images, W_mlp, b_mlp, W_final, b_final):
    p0, p1, p2 = _plane(images, 0), _plane(images, 3), _plane(images, 6)
    tbl = jnp.repeat(
        jnp.zeros((_TBL,), jnp.float32).at[:150].set(W_mlp[0]), 16)
    rs = W_final.sum(axis=1)
    par = (jnp.zeros((8, 16), jnp.float32)
           .at[0].set(rs[0])
           .at[1].set(rs[1])
           .at[2].set(b_final[0])
           .at[3].set(b_final[1])
           .at[4].set(b_mlp[0]))
    return _run(p0, p1, p2, tbl, par)


# speedup vs baseline: 1.2525x; 1.2525x over previous
import functools

import jax
import jax.numpy as jnp
import numpy as np
from jax import lax
from jax.experimental import pallas as pl
from jax.experimental.pallas import tpu as pltpu
from jax.experimental.pallas import tpu_sc as plsc

_N = 1024
_ROW = 1280
_NW = 32
_SPW = _N // _NW
_CHUNK = 16
_TBL = 160

def _sc_body(img_hbm, tbl_hbm, par_hbm, out_hbm,
             buf_a, buf_b, tbl_v, par_v, scr_v, out_v, sem_a, sem_b):
    wid = lax.axis_index("s") * 2 + lax.axis_index("c")
    pltpu.sync_copy(tbl_hbm, tbl_v)
    pltpu.sync_copy(par_hbm, par_v)

    row0 = wid * (2 * 48)
    cps_a = [pltpu.async_copy(img_hbm.at[pl.ds(row0, 48)], buf_a, sem_a)]
    cps_b = [pltpu.async_copy(img_hbm.at[pl.ds(row0 + 48, 48)], buf_b, sem_b)]

    lane = lax.iota(jnp.int32, 16)

    a0 = par_v[0]
    a1 = par_v[1]
    b0 = par_v[2]
    b1 = par_v[3]
    cm = par_v[4]

    def do_chunk(buf, cc):
        def sample_body(s, _):
            acc = jnp.zeros((16,), jnp.float32)
            for r in range(3):
                off2 = jnp.int32(800 * r) + lane

                def body(j, acc):
                    v64 = buf[3 * s + r, pl.ds(j * 64, 64)]
                    w = plsc.bitcast(v64, jnp.int32)
                    for k in range(4):
                        if k == 0:
                            ti = (w << 4) & 0xFF0
                        else:
                            ti = (w >> (8 * k - 4)) & 0xFF0
                        acc = acc + plsc.load_gather(tbl_v, [ti + off2])
                    return acc

                acc = lax.fori_loop(0, _ROW // 64, body, acc, unroll=4)
            plsc.store_scatter(scr_v, [s * 17 + lane], acc)
            return 0

        lax.fori_loop(0, 16, sample_body, 0)

        s_acc = jnp.zeros((16,), jnp.float32)
        for p in range(16):
            s_acc = s_acc + plsc.load_gather(scr_v, [lane * 17 + p])

        mean = s_acc / 10.0 + cm
        orow = jnp.int32(16 * cc) + lane
        plsc.store_scatter(out_v, [orow, jnp.zeros((16,), jnp.int32)],
                           mean * a0 + b0)
        plsc.store_scatter(out_v, [orow, jnp.ones((16,), jnp.int32)],
                           mean * a1 + b1)

    for cp in cps_a:
        cp.wait()
    do_chunk(buf_a, 0)
    for cp in cps_b:
        cp.wait()
    do_chunk(buf_b, 1)

    pltpu.sync_copy(out_v, out_hbm.at[pl.ds(wid * _SPW, _SPW)])


@jax.jit
def _run(img, tbl, par):
    mesh = plsc.VectorSubcoreMesh(core_axis_name="c", subcore_axis_name="s")
    return pl.kernel(
        _sc_body,
        out_type=jax.ShapeDtypeStruct((_N, 2), jnp.float32),
        mesh=mesh,
        compiler_params=pltpu.CompilerParams(use_tc_tiling_on_sc=False,
                                             needs_layout_passes=False),
        scratch_types=[
            pltpu.VMEM((48, _ROW), jnp.uint8),
            pltpu.VMEM((48, _ROW), jnp.uint8),
            pltpu.VMEM((_TBL * 16,), jnp.float32),
            pltpu.VMEM((8, 16), jnp.float32),
            pltpu.VMEM((272,), jnp.float32),
            pltpu.VMEM((_SPW, 2), jnp.float32),
            pltpu.SemaphoreType.DMA,
            pltpu.SemaphoreType.DMA,
        ],
    )(img, tbl, par)


def kernel(images, W_mlp, b_mlp, W_final, b_final):
    t = jnp.transpose(images[:, ::3], (0, 1, 3, 2))
    img = (t * 50.0).astype(jnp.uint8).reshape(_N * 3, _ROW)
    tbl = jnp.repeat(
        jnp.zeros((_TBL,), jnp.float32).at[:150].set(W_mlp[0]), 16)
    rs = W_final.sum(axis=1)
    par = (jnp.zeros((8, 16), jnp.float32)
           .at[0].set(rs[0])
           .at[1].set(rs[1])
           .at[2].set(b_final[0])
           .at[3].set(b_final[1])
           .at[4].set(b_mlp[0]))
    return _run(img, tbl, par)
